# baseline (speedup 1.0000x reference)
import functools

import jax
import jax.numpy as jnp
from jax import lax
from jax.experimental import pallas as pl
from jax.experimental.pallas import tpu as pltpu
from jax.experimental.pallas import tpu_sc as plsc


def _info():
    info = plsc.get_sparse_core_info()
    return info.num_cores, info.num_subcores


@functools.lru_cache(maxsize=None)
def _make_detile(vocab, dim):
    nc, ns = _info()
    num_workers = nc * ns
    block = 1024
    n_fullb = vocab // block
    rem = vocab - n_fullb * block
    base_b, extra_b = divmod(n_fullb, num_workers)
    assert rem % 8 == 0

    mesh = plsc.VectorSubcoreMesh(core_axis_name="c", subcore_axis_name="s")

    @functools.partial(
        pl.kernel,
        out_type=jax.ShapeDtypeStruct((vocab * dim // 128, 128), jnp.float32),
        mesh=mesh,
        compiler_params=pltpu.CompilerParams(
            use_tc_tiling_on_sc=True, needs_layout_passes=False),
        scratch_types=[
            pltpu.VMEM((block, 32), jnp.float32),
            pltpu.VMEM((block, 32), jnp.float32),
            pltpu.SemaphoreType.DMA,
            pltpu.SemaphoreType.DMA,
            pltpu.SemaphoreType.DMA,
            pltpu.SemaphoreType.DMA,
        ],
    )
    def detile_kernel(tab_hbm, out_hbm, b0, b1, si0, si1, so0, so1):
        bufs = [b0, b1]
        si = [si0, si1]
        so = [so0, so1]
        wid = lax.axis_index("s") * nc + lax.axis_index("c")
        my_n = jnp.where(wid < extra_b, base_b + 1, base_b)
        start = wid * base_b + jnp.minimum(wid, extra_b)

        def issue_in(b, g):
            pltpu.async_copy(tab_hbm.at[pl.ds(g * block, block), :],
                             bufs[b], si[b])

        def wait_in(b, g):
            pltpu.make_async_copy(tab_hbm.at[pl.ds(g * block, block), :],
                                  bufs[b], si[b]).wait()

        packs = block * 32 // 128

        def issue_out(b, g):
            pltpu.async_copy(bufs[b].reshape(packs, 128),
                             out_hbm.at[pl.ds(g * packs, packs), :], so[b])

        def wait_out(b, g):
            pltpu.make_async_copy(
                bufs[b].reshape(packs, 128),
                out_hbm.at[pl.ds(g * packs, packs), :], so[b]).wait()

        issue_in(0, start)
        issue_in(1, start + 1)

        def body(j, _):
            for b in range(2):
                g = start + 2 * j + b

                @pl.when(g < start + my_n)
                def _():
                    wait_in(b, g)

                    @pl.when(j > 0)
                    def _():
                        wait_out(b, g - 2)

                    issue_out(b, g)

                    @pl.when(g + 2 < start + my_n)
                    def _():
                        issue_in(b, g + 2)
            return 0

        lax.fori_loop(0, (base_b + 2) // 2, body, 0)
        for b in range(2):
            g_b = start + jnp.where((my_n - 1) % 2 == b, my_n - 1, my_n - 2)
            wait_out(b, g_b)

        if rem:
            @pl.when(wid == num_workers - 1)
            def _():
                pltpu.sync_copy(
                    tab_hbm.at[pl.ds(n_fullb * block, rem), :],
                    bufs[0].at[pl.ds(0, rem), :])
                pltpu.sync_copy(
                    bufs[0].reshape(packs, 128).at[pl.ds(0, rem * 32 // 128), :],
                    out_hbm.at[pl.ds(n_fullb * packs, rem * 32 // 128), :])

    return detile_kernel


@functools.lru_cache(maxsize=None)
def _make_gather(vocab, dim, num_rows):
    nc, ns = _info()
    num_workers = nc * ns
    assert num_rows % (8 * num_workers) == 0
    rows_per_worker = num_rows // num_workers

    chunk = 1600
    while rows_per_worker % chunk:
        chunk //= 2
    n_chunks = rows_per_worker // chunk

    mesh = plsc.VectorSubcoreMesh(core_axis_name="c", subcore_axis_name="s")

    @functools.partial(
        pl.kernel,
        out_type=jax.ShapeDtypeStruct((num_rows, 128), jnp.float32),
        mesh=mesh,
        compiler_params=pltpu.CompilerParams(use_tc_tiling_on_sc=False),
        scratch_types=[
            pltpu.VMEM((rows_per_worker,), jnp.int32),
            pltpu.VMEM((chunk, dim), jnp.float32),
            pltpu.VMEM((chunk, dim), jnp.float32),
            pltpu.SemaphoreType.DMA,
            pltpu.SemaphoreType.DMA,
            pltpu.SemaphoreType.DMA,
            pltpu.SemaphoreType.DMA,
        ],
    )
    def gather_kernel(table_hbm, idx_hbm, out_hbm, idx_v, rows0, rows1,
                      sg0, sg1, sw0, sw1):
        rows = [rows0, rows1]
        sg = [sg0, sg1]
        sw = [sw0, sw1]
        wid = lax.axis_index("s") * nc + lax.axis_index("c")
        base = wid * rows_per_worker

        pltpu.sync_copy(idx_hbm.at[pl.ds(base, rows_per_worker)], idx_v)

        def start_gather(i):
            b = i % 2
            pltpu.async_copy(
                table_hbm.at[idx_v.at[pl.ds(i * chunk, chunk)]], rows[b], sg[b])

        def wait_gather(i):
            b = i % 2
            pltpu.make_async_copy(
                table_hbm.at[idx_v.at[pl.ds(i * chunk, chunk)]], rows[b],
                sg[b]).wait()

        def start_wb(i):
            b = i % 2
            pltpu.async_copy(
                rows[b],
                out_hbm.at[pl.ds(base + i * chunk, chunk), pl.ds(0, dim)],
                sw[b])

        def wait_wb(i):
            b = i % 2
            pltpu.make_async_copy(
                rows[b],
                out_hbm.at[pl.ds(base + i * chunk, chunk), pl.ds(0, dim)],
                sw[b]).wait()

        start_gather(0)
        for i in range(1, n_chunks):
            wait_gather(i - 1)
            start_wb(i - 1)
            if i >= 2:
                wait_wb(i)
            start_gather(i)
        wait_gather(n_chunks - 1)
        start_wb(n_chunks - 1)
        wait_wb(n_chunks - 2)
        wait_wb(n_chunks - 1)

    return gather_kernel


def kernel(x, embedding_matrix):
    batch, hist = x.shape
    vocab, dim = embedding_matrix.shape
    flat_idx = x.reshape(-1)
    gather = _make_gather(vocab, dim, batch * hist)
    out_pad = gather(embedding_matrix, flat_idx)
    return out_pad[:, :dim].reshape(batch, hist, dim)
